# baseline (speedup 1.0000x reference)
import functools

import jax
import jax.numpy as jnp
from jax import lax
from jax.experimental import pallas as pl
from jax.experimental.pallas import tpu as pltpu
from jax.experimental.pallas import tpu_sc as plsc

_B = 1024
_V = 100000
_H = 200
_L = 16

_NC = 2
_NS = 16
_NW = _NC * _NS
_TOTAL = _B * _H
_WIN = 128
_ROWS_W = _B // _NW
_PER_W = _ROWS_W * _H
_CHUNK_E = 640
_NCHUNK = _PER_W // _CHUNK_E
_STREAM = 128
_NSTREAM = _CHUNK_E // _STREAM


def _gather_body(xt_hbm, idx_hbm, out_hbm, idx_v, staged, out_v, sem):
    wid = lax.axis_index("s") * _NC + lax.axis_index("c")
    group = wid // 4
    quarter = wid % 4
    c0 = group * _WIN
    sel0 = quarter * _ROWS_W
    ebase = wid * _PER_W
    pltpu.sync_copy(idx_hbm.at[pl.ds(ebase, _PER_W)], idx_v)

    def chunk_body(ci, carry):
        b_local, rem = carry
        cbase = ci * _CHUNK_E

        for k in range(_NSTREAM):
            pltpu.make_async_copy(
                xt_hbm.at[
                    idx_v.at[pl.ds(cbase + k * _STREAM, _STREAM)],
                    pl.ds(c0, _WIN),
                ],
                staged.at[pl.ds(k * _STREAM, _STREAM)],
                sem,
            ).start()
        for k in range(_NSTREAM):
            pltpu.make_async_copy(
                xt_hbm.at[
                    idx_v.at[pl.ds(cbase + k * _STREAM, _STREAM)],
                    pl.ds(c0, _WIN),
                ],
                staged.at[pl.ds(k * _STREAM, _STREAM)],
                sem,
            ).wait()

        def sel_body(t, scarry):
            bl, srem = scarry
            lanes = lax.iota(jnp.int32, _L)
            bump = jnp.where(lanes >= (_H - srem), 1, 0).astype(jnp.int32)
            sel = sel0 + bl + bump
            evec = lanes + t * _L
            val = plsc.load_gather(staged, [evec, sel])
            out_v[pl.ds(cbase + t * _L, _L)] = val
            srem2 = srem + _L
            wrap = srem2 >= _H
            return (
                jnp.where(wrap, bl + 1, bl),
                jnp.where(wrap, srem2 - _H, srem2),
            )

        return lax.fori_loop(0, _CHUNK_E // _L, sel_body, (b_local, rem))

    lax.fori_loop(0, _NCHUNK, chunk_body, (jnp.int32(0), jnp.int32(0)))
    pltpu.sync_copy(out_v, out_hbm.at[pl.ds(ebase, _PER_W)])


@functools.partial(
    pl.kernel,
    out_type=jax.ShapeDtypeStruct((_TOTAL,), jnp.float32),
    mesh=plsc.VectorSubcoreMesh(core_axis_name="c", subcore_axis_name="s"),
    compiler_params=pltpu.CompilerParams(needs_layout_passes=False),
    scratch_types=[
        pltpu.VMEM((_PER_W,), jnp.int32),
        pltpu.VMEM((_CHUNK_E, _WIN), jnp.float32),
        pltpu.VMEM((_PER_W,), jnp.float32),
        pltpu.SemaphoreType.DMA,
    ],
)
def _sc_gather(xt_hbm, idx_hbm, out_hbm, idx_v, staged, out_v, sem):
    _gather_body(xt_hbm, idx_hbm, out_hbm, idx_v, staged, out_v, sem)


def kernel(x, dim, index, sparse_grad):
    del dim, sparse_grad
    idx_flat = index.astype(jnp.int32).reshape(_TOTAL)
    out = _sc_gather(x.T, idx_flat)
    return out.reshape(_B, _H)


# speedup vs baseline: 2.2277x; 2.2277x over previous
import functools

import jax
import jax.numpy as jnp
from jax import lax
from jax.experimental import pallas as pl
from jax.experimental.pallas import tpu as pltpu
from jax.experimental.pallas import tpu_sc as plsc

_B = 1024
_V = 100000
_H = 200
_L = 16

_NC = 2
_NS = 16
_NW = _NC * _NS
_TOTAL = _B * _H
_XN = _B * _V
_PER_W = _TOTAL // _NW
_CHUNK = 128
_STREAMS = _PER_W // _CHUNK
_FIRE = 10


def _gather_body(x1d_hbm, idx_hbm, out_hbm, idx_v, out_v, sem):
    wid = lax.axis_index("s") * _NC + lax.axis_index("c")
    gbase = wid * _PER_W
    pltpu.sync_copy(idx_hbm.at[pl.ds(gbase, _PER_W)], idx_v)

    def off_body(t, carry):
        sl = pl.ds(t * _L, _L)
        lanes = lax.iota(jnp.int32, _L)
        p0 = gbase + t * _L
        bt = (p0 >> 10) & 7
        bc0 = p0 & 127
        v = idx_v[sl]
        off = (
            ((v >> 3) << 13)
            + ((v & 7) << 7)
            + ((bt << 10) + bc0)
            + lanes
        )
        idx_v[sl] = off
        return carry

    lax.fori_loop(0, _PER_W // _L, off_body, 0)

    def fire_drain(g, carry):
        base = g * _FIRE * _CHUNK
        for k in range(_FIRE):
            sl = pl.ds(base + k * _CHUNK, _CHUNK)
            pltpu.make_async_copy(
                x1d_hbm.at[idx_v.at[sl]], out_v.at[sl], sem
            ).start()
        for k in range(_FIRE):
            sl = pl.ds(base + k * _CHUNK, _CHUNK)
            pltpu.make_async_copy(
                x1d_hbm.at[idx_v.at[sl]], out_v.at[sl], sem
            ).wait()
        return carry

    lax.fori_loop(0, _STREAMS // _FIRE, fire_drain, 0)

    pltpu.sync_copy(out_v, out_hbm.at[pl.ds(gbase, _PER_W)])


@functools.partial(
    pl.kernel,
    out_type=jax.ShapeDtypeStruct((_TOTAL,), jnp.float32),
    mesh=plsc.VectorSubcoreMesh(core_axis_name="c", subcore_axis_name="s"),
    compiler_params=pltpu.CompilerParams(needs_layout_passes=False),
    scratch_types=[
        pltpu.VMEM((_PER_W,), jnp.int32),
        pltpu.VMEM((_PER_W,), jnp.float32),
        pltpu.SemaphoreType.DMA,
    ],
)
def _sc_gather(x1d_hbm, idx_hbm, out_hbm, idx_v, out_v, sem):
    _gather_body(x1d_hbm, idx_hbm, out_hbm, idx_v, out_v, sem)


def kernel(x, dim, index, sparse_grad):
    del dim, sparse_grad
    x1d = x.T.reshape(12500, 8, 8, 128).transpose(0, 2, 1, 3).reshape(_XN)
    idx1d = (
        index.T.reshape(25, 8, 8, 128).transpose(0, 2, 1, 3).reshape(_TOTAL)
    ).astype(jnp.int32)
    out1d = _sc_gather(x1d, idx1d)
    return out1d.reshape(25, 8, 8, 128).transpose(0, 2, 1, 3).reshape(_H, _B).T


# speedup vs baseline: 2.4176x; 1.0853x over previous
import functools

import jax
import jax.numpy as jnp
from jax import lax
from jax.experimental import pallas as pl
from jax.experimental.pallas import tpu as pltpu
from jax.experimental.pallas import tpu_sc as plsc

_B = 1024
_V = 100000
_H = 200
_L = 16

_NC = 2
_NS = 16
_NW = _NC * _NS
_TOTAL = _B * _H
_XN = _B * _V
_PER_W = _TOTAL // _NW
_CHUNK = 128
_STREAMS = _PER_W // _CHUNK
_FIRE = 10


def _gather_body(x1d_hbm, idx_hbm, out_hbm, idx_v, out_v, sem):
    wid = lax.axis_index("s") * _NC + lax.axis_index("c")
    gbase = wid * _PER_W
    pltpu.sync_copy(idx_hbm.at[pl.ds(gbase, _PER_W)], idx_v)

    def one_off(t):
        sl = pl.ds(t * _L, _L)
        lanes = lax.iota(jnp.int32, _L)
        p0 = gbase + t * _L
        bt = (p0 >> 10) & 7
        bc0 = p0 & 127
        v = idx_v[sl]
        idx_v[sl] = (
            ((v >> 3) << 13)
            + ((v & 7) << 7)
            + ((bt << 10) + bc0)
            + lanes
        )

    _GV = _FIRE * _CHUNK // _L

    def off_group(vbase, carry):
        def off_body(u, c):
            for j in range(4):
                one_off(vbase + u * 4 + j)
            return c
        return lax.fori_loop(0, _GV // 4, off_body, carry)

    off_group(0, 0)

    def fire_drain(g, carry):
        base = g * _FIRE * _CHUNK
        for k in range(_FIRE):
            sl = pl.ds(base + k * _CHUNK, _CHUNK)
            pltpu.make_async_copy(
                x1d_hbm.at[idx_v.at[sl]], out_v.at[sl], sem
            ).start()
        ng = _STREAMS // _FIRE
        carry = off_group(((g + 1) % ng) * _GV, carry)
        for k in range(_FIRE):
            sl = pl.ds(base + k * _CHUNK, _CHUNK)
            pltpu.make_async_copy(
                x1d_hbm.at[idx_v.at[sl]], out_v.at[sl], sem
            ).wait()
        return carry

    lax.fori_loop(0, _STREAMS // _FIRE, fire_drain, 0)

    pltpu.sync_copy(out_v, out_hbm.at[pl.ds(gbase, _PER_W)])


@functools.partial(
    pl.kernel,
    out_type=jax.ShapeDtypeStruct((_TOTAL,), jnp.float32),
    mesh=plsc.VectorSubcoreMesh(core_axis_name="c", subcore_axis_name="s"),
    compiler_params=pltpu.CompilerParams(needs_layout_passes=False),
    scratch_types=[
        pltpu.VMEM((_PER_W,), jnp.int32),
        pltpu.VMEM((_PER_W,), jnp.float32),
        pltpu.SemaphoreType.DMA,
    ],
)
def _sc_gather(x1d_hbm, idx_hbm, out_hbm, idx_v, out_v, sem):
    _gather_body(x1d_hbm, idx_hbm, out_hbm, idx_v, out_v, sem)


def kernel(x, dim, index, sparse_grad):
    del dim, sparse_grad
    x1d = x.T.reshape(12500, 8, 8, 128).transpose(0, 2, 1, 3).reshape(_XN)
    idx1d = (
        index.T.reshape(25, 8, 8, 128).transpose(0, 2, 1, 3).reshape(_TOTAL)
    ).astype(jnp.int32)
    out1d = _sc_gather(x1d, idx1d)
    return out1d.reshape(25, 8, 8, 128).transpose(0, 2, 1, 3).reshape(_H, _B).T
